# baseline (speedup 1.0000x reference)
import functools

import jax
import jax.numpy as jnp
from jax import lax
from jax.experimental import pallas as pl
from jax.experimental.pallas import tpu as pltpu
from jax.experimental.pallas import tpu_sc as plsc

N_NODES = 10000
N_EDGES = 320000
D = 128

NC = 2
NS = 16
NW = NC * NS
EW = N_EDGES // NW
C = 80
SE = 2000
NSUP = EW // SE
CS = SE // C

RPT = 632
N_PAD = RPT * NS


def _make_sc_scatter():
    mesh = plsc.VectorSubcoreMesh(core_axis_name="c", subcore_axis_name="s")

    @functools.partial(
        pl.kernel,
        mesh=mesh,
        compiler_params=pltpu.CompilerParams(needs_layout_passes=False,
                                             use_tc_tiling_on_sc=False),
        out_type=jax.ShapeDtypeStruct((NC, N_PAD, D), jnp.float32),
        scratch_types=[
            pltpu.VMEM((SE,), jnp.int32),
            pltpu.VMEM((SE,), jnp.int32),
            pltpu.VMEM((SE,), jnp.float32),
            pltpu.VMEM((C, D // 2), jnp.uint32),
            pltpu.VMEM((C, D // 2), jnp.uint32),
            pltpu.VMEM((C, D // 2), jnp.uint32),
            pltpu.VMEM((C, D // 2), jnp.uint32),
            pltpu.VMEM((C, D), jnp.float32),
            pltpu.VMEM((C, D), jnp.float32),
            pltpu.VMEM((C,), jnp.int32),
            pltpu.VMEM((C,), jnp.int32),
            pltpu.VMEM_SHARED((N_PAD, D), jnp.float32),
            pltpu.SemaphoreType.DMA,
            pltpu.SemaphoreType.DMA,
            pltpu.SemaphoreType.DMA,
            pltpu.SemaphoreType.DMA,
            pltpu.SemaphoreType.DMA,
            pltpu.SemaphoreType.DMA,
        ],
    )
    def sc_scatter(table, src, dst, w, out,
                   srcb, dstb, wb, g0, g1, g2, g3, f0, f1, d0, d1, acc,
                   g0s, g1s, g2s, g3s, s0s, s1s):
        c = lax.axis_index("c")
        s = lax.axis_index("s")
        wid = c * NS + s

        gbufs = (g0, g1, g2, g3)
        fbufs = (f0, f1)
        dvs = (d0, d1)
        gsems = (g0s, g1s, g2s, g3s)
        ssems = (s0s, s1s)

        zv = jnp.zeros((16,), jnp.float32)

        def zrow(e, cc):
            for k in range(D // 16):
                f0[e, pl.ds(k * 16, 16)] = zv
            return cc

        lax.fori_loop(0, 79, zrow, 0)
        for k in range(8):
            pltpu.sync_copy(f0.at[pl.ds(0, 79)],
                            acc.at[pl.ds(s * RPT + k * 79, 79)])
        plsc.subcore_barrier()

        base = wid * EW

        def gather_start(goff, b):
            pltpu.async_copy(table.at[srcb.at[pl.ds(goff, C)]], gbufs[b],
                             gsems[b])

        def gather_wait(goff, b):
            pltpu.make_async_copy(table.at[srcb.at[pl.ds(goff, C)]], gbufs[b],
                                  gsems[b]).wait()

        def scatter_wait(b):
            pltpu.make_async_copy(fbufs[b], acc.at[dvs[b]], ssems[b]).wait()

        def process(goff, bg, bs):
            rb = gbufs[bg]
            rf = fbufs[bs]
            dv = dvs[bs]
            for k in range(C // 16):
                dv[pl.ds(k * 16, 16)] = dstb[pl.ds(goff + k * 16, 16)]

            def scale(i, cc):
                wv = wb[pl.ds(goff + i * 16, 16)]
                for j in range(16):
                    e = i * 16 + j
                    we = wv[j]
                    for k in range(D // 32):
                        pu = rb[e, pl.ds(k * 16, 16)]
                        a = plsc.bitcast(pu << jnp.uint32(16), jnp.float32)
                        b = plsc.bitcast(pu & jnp.uint32(0xFFFF0000),
                                         jnp.float32)
                        rf[e, pl.ds(k * 32, 16)] = a * we
                        rf[e, pl.ds(k * 32 + 16, 16)] = b * we
                return cc

            lax.fori_loop(0, C // 16, scale, 0)
            pltpu.async_copy(rf, acc.at[dv], ssems[bs], add=True)

        def slot(g, j):
            bg = j % 4
            bs = j % 2

            @pl.when(g + 2 < CS)
            def _():
                gather_start((g + 2) * C, (j + 2) % 4)

            gather_wait(g * C, bg)

            @pl.when(g >= 2)
            def _():
                scatter_wait(bs)

            process(g * C, bg, bs)

        def super_chunk(sc, carry):
            off = pl.multiple_of(base + sc * SE, 8)
            pltpu.sync_copy(src.at[pl.ds(off, SE)], srcb)
            pltpu.sync_copy(dst.at[pl.ds(off, SE)], dstb)
            pltpu.sync_copy(w.at[pl.ds(off, SE)], wb)
            gather_start(0, 0)
            gather_start(C, 1)

            def quad(i, cc):
                for j in range(4):
                    slot(4 * i + j, j)
                return cc

            lax.fori_loop(0, CS // 4, quad, 0)
            slot(CS - 1, 0)
            scatter_wait(1)
            scatter_wait(0)
            return carry

        lax.fori_loop(0, NSUP, super_chunk, 0)
        plsc.subcore_barrier()

        r0 = s * RPT
        pltpu.sync_copy(acc.at[pl.ds(r0, RPT)], out.at[c, pl.ds(r0, RPT)])

    return sc_scatter


_sc_scatter = _make_sc_scatter()


def _interleave_bf16(t):
    n = t.shape[0]
    return (t.reshape(n, D // 32, 2, 16).swapaxes(2, 3)
            .reshape(n, D).astype(jnp.bfloat16))


def _pack_u32(t_bf16):
    n = t_bf16.shape[0]
    return jax.lax.bitcast_convert_type(
        t_bf16.reshape(n, D // 2, 2), jnp.uint32)


def _affine_body(p_ref, x_ref, wr_ref, wt_ref, b_ref, o_ref, obf_ref):
    acc = p_ref[0] + p_ref[1]
    t = lax.dot_general(acc, wr_ref[...], (((1,), (1,)), ((), ())),
                        preferred_element_type=jnp.float32)
    t = t + lax.dot_general(x_ref[...], wt_ref[...], (((1,), (1,)), ((), ())),
                            preferred_element_type=jnp.float32)
    h = jax.nn.sigmoid(t + b_ref[...])
    o_ref[...] = h
    obf_ref[...] = _interleave_bf16(h)


def _affine_sigmoid(partials, x, w_rel, w_root, bias2d):
    blk = 2000
    grid = N_NODES // blk
    return pl.pallas_call(
        _affine_body,
        grid=(grid,),
        in_specs=[
            pl.BlockSpec((NC, blk, D), lambda i: (0, i, 0)),
            pl.BlockSpec((blk, D), lambda i: (i, 0)),
            pl.BlockSpec((D, D), lambda i: (0, 0)),
            pl.BlockSpec((D, D), lambda i: (0, 0)),
            pl.BlockSpec((1, D), lambda i: (0, 0)),
        ],
        out_specs=[
            pl.BlockSpec((blk, D), lambda i: (i, 0)),
            pl.BlockSpec((blk, D), lambda i: (i, 0)),
        ],
        out_shape=[
            jax.ShapeDtypeStruct((N_NODES, D), jnp.float32),
            jax.ShapeDtypeStruct((N_NODES, D), jnp.bfloat16),
        ],
    )(partials, x, w_rel, w_root, bias2d)


def kernel(X, edge_index, edge_weight,
           Whx_rel, Whx_b, Whx_root,
           Whh_rel, Whh_b, Whh_root,
           Wy_rel, Wy_b, Wy_root):
    src = edge_index[0]
    dst = edge_index[1]
    xt = _pack_u32(_interleave_bf16(X))

    p1 = _sc_scatter(xt, src, dst, edge_weight)
    b1 = (Whx_b + Whh_b).reshape(1, D)
    H, hbf = _affine_sigmoid(p1, X, Whx_rel, Whx_root, b1)

    p2 = _sc_scatter(_pack_u32(hbf), src, dst, edge_weight)
    b2 = Wy_b.reshape(1, D)
    yt, _ = _affine_sigmoid(p2, H, Wy_rel, Wy_root, b2)
    return yt


# speedup vs baseline: 3.2050x; 3.2050x over previous
import functools

import jax
import jax.numpy as jnp
from jax import lax
from jax.experimental import pallas as pl
from jax.experimental.pallas import tpu as pltpu
from jax.experimental.pallas import tpu_sc as plsc

N_NODES = 10000
N_EDGES = 320000
D = 128

NC = 2
NS = 16
NW = NC * NS
EW = N_EDGES // NW
C = 80
SE = 2000
NSUP = EW // SE
CS = SE // C

RPT = 632
N_PAD = RPT * NS


def _make_sc_scatter():
    mesh = plsc.VectorSubcoreMesh(core_axis_name="c", subcore_axis_name="s")

    @functools.partial(
        pl.kernel,
        mesh=mesh,
        out_type=jax.ShapeDtypeStruct((NC, N_PAD, D), jnp.float32),
        scratch_types=[
            pltpu.VMEM((SE,), jnp.int32),
            pltpu.VMEM((SE,), jnp.int32),
            pltpu.VMEM((SE,), jnp.float32),
            pltpu.VMEM((C, D), jnp.float32),
            pltpu.VMEM((C, D), jnp.float32),
            pltpu.VMEM((C, D), jnp.float32),
            pltpu.VMEM((C, D), jnp.float32),
            pltpu.VMEM((C,), jnp.int32),
            pltpu.VMEM((C,), jnp.int32),
            pltpu.VMEM((C,), jnp.int32),
            pltpu.VMEM((C,), jnp.int32),
            pltpu.VMEM_SHARED((N_PAD, D), jnp.float32),
            pltpu.SemaphoreType.DMA,
            pltpu.SemaphoreType.DMA,
            pltpu.SemaphoreType.DMA,
            pltpu.SemaphoreType.DMA,
            pltpu.SemaphoreType.DMA,
            pltpu.SemaphoreType.DMA,
            pltpu.SemaphoreType.DMA,
            pltpu.SemaphoreType.DMA,
        ],
    )
    def sc_scatter(table, src, dst, w, out,
                   srcb, dstb, wb, r0, r1, r2, r3, d0, d1, d2, d3, acc,
                   g0s, g1s, g2s, g3s, s0s, s1s, s2s, s3s):
        c = lax.axis_index("c")
        s = lax.axis_index("s")
        wid = c * NS + s

        rows = (r0, r1, r2, r3)
        dvs = (d0, d1, d2, d3)
        gsems = (g0s, g1s, g2s, g3s)
        ssems = (s0s, s1s, s2s, s3s)

        zv = jnp.zeros((16,), jnp.float32)

        def zrow(e, cc):
            for k in range(D // 16):
                r0[e, pl.ds(k * 16, 16)] = zv
            return cc

        lax.fori_loop(0, 79, zrow, 0)
        for k in range(8):
            pltpu.sync_copy(r0.at[pl.ds(0, 79)],
                            acc.at[pl.ds(s * RPT + k * 79, 79)])
        plsc.subcore_barrier()

        base = wid * EW

        def gather_start(goff, b):
            pltpu.async_copy(table.at[srcb.at[pl.ds(goff, C)]], rows[b],
                             gsems[b])

        def gather_wait(goff, b):
            pltpu.make_async_copy(table.at[srcb.at[pl.ds(goff, C)]], rows[b],
                                  gsems[b]).wait()

        def scatter_wait(b):
            pltpu.make_async_copy(rows[b], acc.at[dvs[b]], ssems[b]).wait()

        def process(goff, b):
            rv = rows[b]
            dv = dvs[b]
            for k in range(C // 16):
                dv[pl.ds(k * 16, 16)] = dstb[pl.ds(goff + k * 16, 16)]

            def scale(i, cc):
                wv = wb[pl.ds(goff + i * 16, 16)]
                for j in range(16):
                    e = i * 16 + j
                    we = wv[j]
                    for k in range(D // 16):
                        sl = pl.ds(k * 16, 16)
                        rv[e, sl] = rv[e, sl] * we
                return cc

            lax.fori_loop(0, C // 16, scale, 0)
            pltpu.async_copy(rv, acc.at[dv], ssems[b], add=True)

        def slot(g, j):
            b = j % 4
            b2 = (j + 2) % 4

            @pl.when(g >= 2)
            def _():
                scatter_wait(b2)

            @pl.when(g + 2 < CS)
            def _():
                gather_start((g + 2) * C, b2)

            gather_wait(g * C, b)
            process(g * C, b)

        def super_chunk(sc, carry):
            off = pl.multiple_of(base + sc * SE, 8)
            pltpu.sync_copy(src.at[pl.ds(off, SE)], srcb)
            pltpu.sync_copy(dst.at[pl.ds(off, SE)], dstb)
            pltpu.sync_copy(w.at[pl.ds(off, SE)], wb)
            gather_start(0, 0)
            gather_start(C, 1)

            def quad(i, cc):
                for j in range(4):
                    slot(4 * i + j, j)
                return cc

            lax.fori_loop(0, CS // 4, quad, 0)
            slot(CS - 1, 0)
            scatter_wait(3)
            scatter_wait(0)
            return carry

        lax.fori_loop(0, NSUP, super_chunk, 0)
        plsc.subcore_barrier()

        r0 = s * RPT
        pltpu.sync_copy(acc.at[pl.ds(r0, RPT)], out.at[c, pl.ds(r0, RPT)])

    return sc_scatter


_sc_scatter = _make_sc_scatter()


def _affine_body(p_ref, x_ref, wr_ref, wt_ref, b_ref, o_ref):
    acc = p_ref[0] + p_ref[1]
    t = lax.dot_general(acc, wr_ref[...], (((1,), (1,)), ((), ())),
                        preferred_element_type=jnp.float32)
    t = t + lax.dot_general(x_ref[...], wt_ref[...], (((1,), (1,)), ((), ())),
                            preferred_element_type=jnp.float32)
    o_ref[...] = jax.nn.sigmoid(t + b_ref[...])


def _affine_sigmoid(partials, x, w_rel, w_root, bias2d):
    blk = 2000
    grid = N_NODES // blk
    return pl.pallas_call(
        _affine_body,
        grid=(grid,),
        in_specs=[
            pl.BlockSpec((NC, blk, D), lambda i: (0, i, 0)),
            pl.BlockSpec((blk, D), lambda i: (i, 0)),
            pl.BlockSpec((D, D), lambda i: (0, 0)),
            pl.BlockSpec((D, D), lambda i: (0, 0)),
            pl.BlockSpec((1, D), lambda i: (0, 0)),
        ],
        out_specs=pl.BlockSpec((blk, D), lambda i: (i, 0)),
        out_shape=jax.ShapeDtypeStruct((N_NODES, D), jnp.float32),
    )(partials, x, w_rel, w_root, bias2d)


def kernel(X, edge_index, edge_weight,
           Whx_rel, Whx_b, Whx_root,
           Whh_rel, Whh_b, Whh_root,
           Wy_rel, Wy_b, Wy_root):
    src = edge_index[0]
    dst = edge_index[1]
    p1 = _sc_scatter(X, src, dst, edge_weight)
    b1 = (Whx_b + Whh_b).reshape(1, D)
    H = _affine_sigmoid(p1, X, Whx_rel, Whx_root, b1)

    p2 = _sc_scatter(H, src, dst, edge_weight)
    b2 = Wy_b.reshape(1, D)
    yt = _affine_sigmoid(p2, H, Wy_rel, Wy_root, b2)
    return yt
